# baseline (speedup 1.0000x reference)
import functools

import jax
import jax.numpy as jnp
from jax import lax
from jax.experimental import pallas as pl
from jax.experimental.pallas import tpu as pltpu
from jax.experimental.pallas import tpu_sc as plsc

_N_NODES = 10000
_N_EDGES = 320000
_D_FEAT = 128
_D_EDGE = 4

_NC = 2
_NS = 16
_NW = _NC * _NS
_RW = 128
_NBLK = _N_EDGES // _RW
_R_LO = _NBLK // _NW
_R_HI = _R_LO + 1
_N_HI = _NBLK - _NW * _R_LO
_N_PAD = 10240
_ZPT = _N_PAD // _NS

_sc_mesh = plsc.VectorSubcoreMesh(
    core_axis_name="c", subcore_axis_name="s", num_cores=_NC, num_subcores=_NS
)


@functools.partial(
    pl.kernel,
    out_type=jax.ShapeDtypeStruct((_NC, _N_PAD), jnp.float32),
    mesh=_sc_mesh,
    scratch_types=[
        pltpu.VMEM((2, _R_HI * _RW), jnp.int32),
        pltpu.VMEM((_D_EDGE, _R_HI * _RW), jnp.float32),
        pltpu.VMEM((_N_PAD,), jnp.float32),
        pltpu.VMEM((_NS, _ZPT), jnp.float32),
        pltpu.VMEM((_ZPT,), jnp.float32),
        pltpu.VMEM_SHARED((_NS, _N_PAD), jnp.float32),
        pltpu.SemaphoreType.DMA,
        pltpu.SemaphoreType.DMA,
    ],
    compiler_params=pltpu.CompilerParams(needs_layout_passes=False),
)
def _sc_segment_sum(
    eij_hbm, ea_hbm, out_hbm, rawidx_v, rawea_v, acc_v, blk_v, red_v, grid_sh,
    sem1, sem2,
):
    c = lax.axis_index("c")
    s = lax.axis_index("s")
    wid = s * _NC + c
    blk0 = _R_LO * wid + jnp.minimum(wid, _N_HI)
    base = blk0 * _RW
    nrows = jnp.where(wid < _N_HI, _R_HI, _R_LO)

    @pl.when(wid < _N_HI)
    def _():
        d1 = pltpu.async_copy(
            eij_hbm.at[:, pl.ds(base, _R_HI * _RW)], rawidx_v, sem1
        )
        d2 = pltpu.async_copy(
            ea_hbm.at[:, pl.ds(base, _R_HI * _RW)], rawea_v, sem2
        )
        d1.wait()
        d2.wait()

    @pl.when(wid >= _N_HI)
    def _():
        d1 = pltpu.async_copy(
            eij_hbm.at[:, pl.ds(base, _R_LO * _RW)],
            rawidx_v.at[:, pl.ds(0, _R_LO * _RW)],
            sem1,
        )
        d2 = pltpu.async_copy(
            ea_hbm.at[:, pl.ds(base, _R_LO * _RW)],
            rawea_v.at[:, pl.ds(0, _R_LO * _RW)],
            sem2,
        )
        d1.wait()
        d2.wait()

    zeros16 = jnp.zeros((16,), jnp.float32)

    def zero(i, carry):
        acc_v[pl.ds(i * 16, 16)] = zeros16
        return carry

    lax.fori_loop(0, _N_PAD // 16, zero, 0)

    def scatter(t, carry):
        idx16 = rawidx_v[0, pl.ds(t * 16, 16)]
        val16 = rawea_v[1, pl.ds(t * 16, 16)]
        plsc.addupdate_scatter(acc_v, [idx16], val16)
        return carry

    lax.fori_loop(0, nrows * (_RW // 16), scatter, 0)

    pltpu.sync_copy(acc_v, grid_sh.at[s])
    plsc.subcore_barrier()

    pltpu.sync_copy(grid_sh.at[:, pl.ds(s * _ZPT, _ZPT)], blk_v)

    for k in range(_NS):
        if k == 0:
            def seed(i, carry):
                red_v[pl.ds(i * 16, 16)] = blk_v[0, pl.ds(i * 16, 16)]
                return carry
            lax.fori_loop(0, _ZPT // 16, seed, 0)
        else:
            def add_row(i, carry, k=k):
                red_v[pl.ds(i * 16, 16)] = (
                    red_v[pl.ds(i * 16, 16)] + blk_v[k, pl.ds(i * 16, 16)]
                )
                return carry
            lax.fori_loop(0, _ZPT // 16, add_row, 0)

    pltpu.sync_copy(red_v, out_hbm.at[c, pl.ds(s * _ZPT, _ZPT)])


_BLK = 1024


def _tc_body(x_ref, p_ref, o_ref):
    pt = jnp.transpose(p_ref[...])
    cbar = pt[:, 0:1] + pt[:, 1:2]
    x = x_ref[...]
    o_ref[:, :_D_FEAT] = x
    o_ref[:, _D_FEAT:] = x * cbar


def _tc_fuse(x, partials):
    return pl.pallas_call(
        _tc_body,
        grid=(pl.cdiv(_N_NODES, _BLK),),
        in_specs=[
            pl.BlockSpec((_BLK, _D_FEAT), lambda i: (i, 0)),
            pl.BlockSpec((2, _BLK), lambda i: (0, i)),
        ],
        out_specs=pl.BlockSpec((_BLK, 2 * _D_FEAT), lambda i: (i, 0)),
        out_shape=jax.ShapeDtypeStruct((_N_NODES, 2 * _D_FEAT), jnp.float32),
    )(x, partials)


def kernel(node_attr, edgeij_pair, edge_attr, g, batch):
    partials = _sc_segment_sum(edgeij_pair, edge_attr.T)
    return _tc_fuse(node_attr, partials)


# speedup vs baseline: 1.0569x; 1.0569x over previous
import functools

import jax
import jax.numpy as jnp
from jax import lax
from jax.experimental import pallas as pl
from jax.experimental.pallas import tpu as pltpu
from jax.experimental.pallas import tpu_sc as plsc

_N_NODES = 10000
_N_EDGES = 320000
_D_FEAT = 128
_D_EDGE = 4

_NC = 2
_NS = 16
_NW = _NC * _NS
_RW = 128
_NBLK = _N_EDGES // _RW
_R_LO = _NBLK // _NW
_R_HI = _R_LO + 1
_N_HI = _NBLK - _NW * _R_LO
_N_PAD = 10240
_ZPT = _N_PAD // _NS

_sc_mesh = plsc.VectorSubcoreMesh(
    core_axis_name="c", subcore_axis_name="s", num_cores=_NC, num_subcores=_NS
)


@functools.partial(
    pl.kernel,
    out_type=jax.ShapeDtypeStruct((_NC, _N_PAD), jnp.float32),
    mesh=_sc_mesh,
    scratch_types=[
        pltpu.VMEM((2, _R_HI * _RW), jnp.int32),
        pltpu.VMEM((_D_EDGE, _R_HI * _RW), jnp.float32),
        pltpu.VMEM((_N_PAD,), jnp.float32),
        pltpu.VMEM((_NS, _ZPT), jnp.float32),
        pltpu.VMEM((_ZPT,), jnp.float32),
        pltpu.VMEM_SHARED((_NS, _N_PAD), jnp.float32),
        pltpu.SemaphoreType.DMA,
        pltpu.SemaphoreType.DMA,
    ],
    compiler_params=pltpu.CompilerParams(needs_layout_passes=False),
)
def _sc_segment_sum(
    eij_hbm, ea_hbm, out_hbm, rawidx_v, rawea_v, acc_v, blk_v, red_v, grid_sh,
    sem1, sem2,
):
    c = lax.axis_index("c")
    s = lax.axis_index("s")
    wid = s * _NC + c
    blk0 = _R_LO * wid + jnp.minimum(wid, _N_HI)
    base = blk0 * _RW
    nrows = jnp.where(wid < _N_HI, _R_HI, _R_LO)

    @pl.when(wid < _N_HI)
    def _():
        d1 = pltpu.async_copy(
            eij_hbm.at[:, pl.ds(base, _R_HI * _RW)], rawidx_v, sem1
        )
        d2 = pltpu.async_copy(
            ea_hbm.at[:, pl.ds(base, _R_HI * _RW)], rawea_v, sem2
        )
        d1.wait()
        d2.wait()

    @pl.when(wid >= _N_HI)
    def _():
        d1 = pltpu.async_copy(
            eij_hbm.at[:, pl.ds(base, _R_LO * _RW)],
            rawidx_v.at[:, pl.ds(0, _R_LO * _RW)],
            sem1,
        )
        d2 = pltpu.async_copy(
            ea_hbm.at[:, pl.ds(base, _R_LO * _RW)],
            rawea_v.at[:, pl.ds(0, _R_LO * _RW)],
            sem2,
        )
        d1.wait()
        d2.wait()

    zeros16 = jnp.zeros((16,), jnp.float32)

    def zero(i, carry):
        acc_v[pl.ds(i * 16, 16)] = zeros16
        return carry

    lax.fori_loop(0, _N_PAD // 16, zero, 0)

    def scatter(t, carry):
        idx16 = rawidx_v[0, pl.ds(t * 16, 16)]
        val16 = rawea_v[1, pl.ds(t * 16, 16)]
        plsc.addupdate_scatter(acc_v, [idx16], val16)
        return carry

    lax.fori_loop(0, nrows * (_RW // 16), scatter, 0)

    pltpu.sync_copy(acc_v, grid_sh.at[s])
    plsc.subcore_barrier()

    pltpu.sync_copy(grid_sh.at[:, pl.ds(s * _ZPT, _ZPT)], blk_v)

    for k in range(_NS):
        if k == 0:
            def seed(i, carry):
                red_v[pl.ds(i * 16, 16)] = blk_v[0, pl.ds(i * 16, 16)]
                return carry
            lax.fori_loop(0, _ZPT // 16, seed, 0)
        else:
            def add_row(i, carry, k=k):
                red_v[pl.ds(i * 16, 16)] = (
                    red_v[pl.ds(i * 16, 16)] + blk_v[k, pl.ds(i * 16, 16)]
                )
                return carry
            lax.fori_loop(0, _ZPT // 16, add_row, 0)

    pltpu.sync_copy(red_v, out_hbm.at[c, pl.ds(s * _ZPT, _ZPT)])


def _tc_body(x_ref, p_ref, o_ref):
    ones = jnp.ones((2, 1), jnp.float32)
    cbar = jax.lax.dot_general(
        p_ref[...], ones, (((0,), (0,)), ((), ())),
        preferred_element_type=jnp.float32,
    )
    x = x_ref[...]
    o_ref[:, :_D_FEAT] = x
    o_ref[:, _D_FEAT:] = x * cbar[:_N_NODES, :]


def _tc_fuse(x, partials):
    return pl.pallas_call(
        _tc_body,
        out_shape=jax.ShapeDtypeStruct((_N_NODES, 2 * _D_FEAT), jnp.float32),
    )(x, partials)


def kernel(node_attr, edgeij_pair, edge_attr, g, batch):
    partials = _sc_segment_sum(edgeij_pair, edge_attr.T)
    return _tc_fuse(node_attr, partials)
